# baseline (speedup 1.0000x reference)
import functools

import jax
import jax.numpy as jnp
from jax import lax
from jax.experimental import pallas as pl
from jax.experimental.pallas import tpu as pltpu

_P = 0.1
_CB = 8
_NB = 4
_CHUNK = 16384


def _mul_body(rand_ref, mask_ref, out_ref):
    keep = (rand_ref[...] >= _P).astype(jnp.float32)
    out_ref[...] = mask_ref[...] * keep[None]


def _make_sc_copy(shape5):
    from jax.experimental.pallas import tpu_sc as plsc

    info = plsc.get_sparse_core_info()
    nw = info.num_cores * info.num_subcores
    C = shape5[0]
    vb_n = shape5[1]
    cpw = C // nw
    nch = cpw * vb_n
    mesh = plsc.VectorSubcoreMesh(core_axis_name="c", subcore_axis_name="s")

    @functools.partial(
        pl.kernel,
        mesh=mesh,
        out_type=jax.ShapeDtypeStruct(shape5, jnp.float32),
        scratch_types=[
            pltpu.VMEM((_NB,) + shape5[2:], jnp.float32),
            pltpu.SemaphoreType.DMA((_NB,)),
            pltpu.SemaphoreType.DMA((_NB,)),
        ],
    )
    def sc_copy(x_hbm, out_hbm, buf, sin, sout):
        w = lax.axis_index("s") * info.num_cores + lax.axis_index("c")
        base_c = w * cpw

        def in_dma(i, slot):
            ci, vb = i // vb_n, i % vb_n
            return pltpu.make_async_copy(
                x_hbm.at[base_c + ci, vb],
                buf.at[slot],
                sin.at[slot],
            )

        def out_dma(i, slot):
            ci, vb = i // vb_n, i % vb_n
            return pltpu.make_async_copy(
                buf.at[slot],
                out_hbm.at[base_c + ci, vb],
                sout.at[slot],
            )

        for s in range(_NB):
            in_dma(s, s).start()
        for i in range(nch):
            slot = i % _NB
            in_dma(i, slot).wait()
            out_dma(i, slot).start()
            nxt = i + _NB
            if nxt < nch:
                out_dma(i, slot).wait()
                in_dma(nxt, slot).start()
        for i in range(nch - _NB, nch):
            out_dma(i, i % _NB).wait()

    return sc_copy


def kernel(x, mask, rand_t):
    C, T, V = mask.shape
    mask_t = jnp.transpose(mask, (0, 2, 1))
    x5 = (jnp.transpose(x, (0, 2, 1))
          .reshape(C, V // 8, 8, T // 128, 128)
          .transpose(0, 1, 3, 2, 4))

    xout5 = _make_sc_copy(x5.shape)(x5)

    blk = pl.BlockSpec((_CB, V, T), lambda i: (i, 0, 0))
    out_t = pl.pallas_call(
        _mul_body,
        grid=(C // _CB,),
        in_specs=[
            pl.BlockSpec((1, T), lambda i: (0, 0)),
            blk,
        ],
        out_specs=blk,
        out_shape=jax.ShapeDtypeStruct((C, V, T), jnp.float32),
    )(rand_t.reshape(1, T), mask_t)

    x_out = jnp.transpose(
        xout5.transpose(0, 1, 3, 2, 4).reshape(C, V, T), (0, 2, 1))
    return (x_out, jnp.transpose(out_t, (0, 2, 1)))


# speedup vs baseline: 1.2340x; 1.2340x over previous
import jax
import jax.numpy as jnp
from jax.experimental import pallas as pl

_P = 0.1
_CB = 8


def _body(rand_ref, mask_ref, x_ref, out_ref, xout_ref):
    keep = (rand_ref[...] >= _P).astype(jnp.float32)
    out_ref[...] = mask_ref[...] * keep[None]
    xout_ref[...] = x_ref[...]


def kernel(x, mask, rand_t):
    C, T, V = mask.shape
    mask_t = jnp.transpose(mask, (0, 2, 1))
    x_t = jnp.transpose(x, (0, 2, 1))

    blk = pl.BlockSpec((_CB, V, T), lambda i: (i, 0, 0))
    out_t, xout_t = pl.pallas_call(
        _body,
        grid=(C // _CB,),
        in_specs=[
            pl.BlockSpec((1, T), lambda i: (0, 0)),
            blk,
            blk,
        ],
        out_specs=[blk, blk],
        out_shape=[
            jax.ShapeDtypeStruct((C, V, T), jnp.float32),
            jax.ShapeDtypeStruct((C, V, T), jnp.float32),
        ],
    )(rand_t.reshape(1, T), mask_t, x_t)
    return (jnp.transpose(xout_t, (0, 2, 1)), jnp.transpose(out_t, (0, 2, 1)))
